# baseline (device time: 118013 ns/iter reference)
import jax
import jax.numpy as jnp
from jax import lax
from jax.experimental import pallas as pl
from jax.experimental.pallas import tpu as pltpu

ROWS = 4096
COLS = 1024
CHUNK = 256
N_CHUNKS = ROWS // CHUNK
W = 768


def kernel(x, dest):
    my_x = lax.axis_index("x")

    keep = (dest == my_x).astype(jnp.int32)
    n_keep = jnp.sum(keep)
    ns = ROWS - n_keep
    order = jnp.argsort(keep, stable=True).astype(jnp.int32)
    ar = jnp.arange(ROWS, dtype=jnp.int32)
    send_ord = jnp.where(ar < ns, order, order[ns - 1])
    keep_ord = jnp.where(ar < n_keep, jnp.roll(order, -ns), order[ROWS - 1])

    def starts_of(ord_arr):
        mins = ord_arr.reshape(N_CHUNKS, CHUNK).min(axis=1)
        return jnp.minimum((mins // 8) * 8, ROWS - W)

    scal = jnp.concatenate(
        [ns.reshape(1), starts_of(send_ord), starts_of(keep_ord)]
    ).astype(jnp.int32)
    xb = x.astype(jnp.bfloat16)

    def body(scal_ref, sord_ref, kord_ref, xb_ref, out_ref,
             buf_ref, kbuf_ref, recv_ref, send_sems, recv_sems):
        mx = lax.axis_index("x")
        my = lax.axis_index("y")
        mz = lax.axis_index("z")
        peer = (1 - mx, my, mz)

        ns = scal_ref[0]
        nk = ROWS - ns
        send_chunks = (ns + CHUNK - 1) // CHUNK
        keep_chunks = (nk + CHUNK - 1) // CHUNK

        barrier_sem = pltpu.get_barrier_semaphore()
        pl.semaphore_signal(
            barrier_sem, inc=1, device_id=peer,
            device_id_type=pl.DeviceIdType.MESH,
        )
        pl.semaphore_wait(barrier_sem, 1)

        def chunk_rdma(i):
            return pltpu.make_async_remote_copy(
                src_ref=buf_ref.at[pl.ds(i * CHUNK, CHUNK), :],
                dst_ref=recv_ref.at[pl.ds(i * CHUNK, CHUNK), :],
                send_sem=send_sems.at[i],
                recv_sem=recv_sems.at[i],
                device_id=peer,
                device_id_type=pl.DeviceIdType.MESH,
            )

        def compact_chunk(c, ord_ref, start, dst_ref):
            ords = ord_ref[pl.ds(c * CHUNK, CHUNK), :]
            col = lax.broadcasted_iota(jnp.int32, (CHUNK, W), 1) + start
            p = (ords == col).astype(jnp.bfloat16)
            rows = lax.dot_general(
                p, xb_ref[pl.ds(start, W), :], (((1,), (0,)), ((), ())),
                preferred_element_type=jnp.float32,
            )
            dst_ref[pl.ds(c * CHUNK, CHUNK), :] = rows.astype(jnp.bfloat16)

        for c in range(N_CHUNKS):
            @pl.when(c < send_chunks)
            def _(c=c):
                start = pl.multiple_of(scal_ref[1 + c], 8)
                compact_chunk(c, sord_ref, start, buf_ref)
                chunk_rdma(c).start()

        for c in range(N_CHUNKS):
            @pl.when(c < keep_chunks)
            def _(c=c):
                start = pl.multiple_of(scal_ref[1 + N_CHUNKS + c], 8)
                compact_chunk(c, kord_ref, start, kbuf_ref)

        for c in range(N_CHUNKS):
            @pl.when(c < send_chunks)
            def _(c=c):
                chunk_rdma(c).wait_recv()

        for c in range(N_CHUNKS):
            @pl.when(c < send_chunks)
            def _(c=c):
                chunk_rdma(c).wait_send()

        keep_base = mx * ns
        recv_base = (1 - mx) * nk
        row = lax.broadcasted_iota(jnp.int32, (ROWS, 1), 0)
        in_keep = (row >= keep_base) & (row < keep_base + nk)
        SLAB = 256
        for s in range(COLS // SLAB):
            cols = pl.ds(s * SLAB, SLAB)
            rolled_keep = pltpu.roll(kbuf_ref[:, cols], keep_base, 0)
            rolled_recv = pltpu.roll(recv_ref[:, cols], recv_base, 0)
            out_ref[:, cols] = jnp.where(in_keep, rolled_keep, rolled_recv)

    return pl.pallas_call(
        body,
        out_shape=jax.ShapeDtypeStruct((ROWS, COLS), jnp.bfloat16),
        in_specs=[
            pl.BlockSpec(memory_space=pltpu.SMEM),
            pl.BlockSpec(memory_space=pltpu.VMEM),
            pl.BlockSpec(memory_space=pltpu.VMEM),
            pl.BlockSpec(memory_space=pltpu.VMEM),
        ],
        out_specs=pl.BlockSpec(memory_space=pltpu.VMEM),
        scratch_shapes=[
            pltpu.VMEM((ROWS, COLS), jnp.bfloat16),
            pltpu.VMEM((ROWS, COLS), jnp.bfloat16),
            pltpu.VMEM((ROWS, COLS), jnp.bfloat16),
            pltpu.SemaphoreType.DMA((N_CHUNKS,)),
            pltpu.SemaphoreType.DMA((N_CHUNKS,)),
        ],
        compiler_params=pltpu.CompilerParams(
            collective_id=0, vmem_limit_bytes=100 * 1024 * 1024
        ),
    )(scal, send_ord.reshape(ROWS, 1), keep_ord.reshape(ROWS, 1), xb)
